# baseline (speedup 1.0000x reference)
import functools

import jax
import jax.numpy as jnp
from jax import lax
from jax.experimental import pallas as pl
from jax.experimental.pallas import tpu as pltpu
from jax.experimental.pallas import tpu_sc as plsc

_BATCH = 16384
_D_STATE = 512
_D_OUT = 512
_EMBED = 16

_NW = 32
_BPW = _BATCH // _NW
_G = 16
_NG = _BPW // _G
_NBUF = 3

_TB = 1024
_CB = 4096


_N_ACT = 1000000
_CL = 2048
_BASE_CH = 15
_BASE_SPAN = _BASE_CH * _CL
_EXTRA0 = _NW * _BASE_SPAN
_PARTIAL = _EXTRA0 + 8 * _CL
_HCAP = 1024
_OPAD = _BATCH + 16


def _sc_gather_rows(table_t, idx):
    mesh = plsc.VectorSubcoreMesh(core_axis_name="c", subcore_axis_name="s")

    @functools.partial(
        pl.kernel,
        out_type=jax.ShapeDtypeStruct((_OPAD, _EMBED), table_t.dtype),
        mesh=mesh,
        compiler_params=pltpu.CompilerParams(
            use_tc_tiling_on_sc=True, needs_layout_passes=False
        ),
        scratch_types=[
            pltpu.VMEM((_BATCH,), jnp.int32),
            pltpu.VMEM((_HCAP,), jnp.int32),
            pltpu.VMEM((_HCAP,), jnp.int32),
            pltpu.VMEM((_HCAP,), jnp.int32),
            pltpu.VMEM((_HCAP,), jnp.int32),
            pltpu.VMEM((2, _EMBED, _CL), jnp.float32),
            pltpu.VMEM((2, 16, _EMBED), jnp.float32),
            pltpu.SMEM((32,), jnp.int32),
            pltpu.SemaphoreType.DMA,
            pltpu.SemaphoreType.DMA,
            pltpu.SemaphoreType.DMA,
            pltpu.SemaphoreType.DMA,
        ],
    )
    def run(tab_hbm, idx_hbm, out_hbm, idx_v, ha, hp, sa, sp, chunks, ring,
            meta, sem0, sem1, osem0, osem1):
        wid = lax.axis_index("s") * 2 + lax.axis_index("c")
        lo = wid * _BASE_SPAN
        elo = _EXTRA0 + wid * _CL
        n_chunks = jnp.where(wid < 9, _BASE_CH + 1, _BASE_CH)
        sems_f = (sem0, sem1)
        osems = (osem0, osem1)
        lane_iota = lax.iota(jnp.int32, 16)

        pltpu.async_copy(idx_hbm, idx_v, sem0).wait()

        dump = jnp.full((16,), _BATCH, jnp.int32)

        @pl.loop(0, _HCAP // 16)
        def _(i):
            sp[pl.ds(i * 16, 16)] = dump
            hp[pl.ds(i * 16, 16)] = dump

        def scan_body(j, cnt):
            v = idx_v[pl.ds(j * 16, 16)]
            pos = lane_iota + j * 16
            m = ((v >= lo) & (v < lo + _BASE_SPAN)) | (
                (v >= elo) & (v < jnp.minimum(elo + _CL, _N_ACT))
            )
            plsc.store_compressed(ha.at[pl.ds(cnt, 16)], v, mask=m)
            plsc.store_compressed(hp.at[pl.ds(cnt, 16)], pos, mask=m)
            return cnt + plsc.all_reduce_population_count(m)[0]

        cnt = lax.fori_loop(0, _BATCH // 16, scan_body, jnp.int32(0))
        nh16 = (cnt + 15) >> 4

        def bucket(c, scnt):
            c_lo = jnp.where(c < _BASE_CH, lo + c * _CL, elo)
            meta[2 * c] = scnt

            def bb(t, s):
                hv = ha[pl.ds(t * 16, 16)]
                pv = hp[pl.ds(t * 16, 16)]
                valid = lane_iota + t * 16 < cnt
                m = (hv >= c_lo) & (hv < c_lo + _CL) & valid
                plsc.store_compressed(sa.at[pl.ds(s, 16)], hv, mask=m)
                plsc.store_compressed(sp.at[pl.ds(s, 16)], pv, mask=m)
                return s + plsc.all_reduce_population_count(m)[0]

            scnt = lax.fori_loop(0, nh16, bb, scnt)
            meta[2 * c + 1] = scnt - meta[2 * c]
            return (scnt + 15) & ~15

        lax.fori_loop(0, 16, bucket, jnp.int32(0))

        def fetch_chunk(c, buf):
            c_lo = jnp.where(c < _BASE_CH, lo + c * _CL, elo)

            @pl.when(c_lo != _PARTIAL)
            def _():
                pltpu.make_async_copy(
                    tab_hbm.at[:, pl.ds(pl.multiple_of(c_lo, 128), _CL)],
                    chunks.at[buf],
                    sems_f[buf],
                ).start()

            @pl.when(c_lo == _PARTIAL)
            def _():
                pltpu.make_async_copy(
                    tab_hbm.at[:, pl.ds(pl.multiple_of(c_lo, 128), 512)],
                    chunks.at[buf, :, pl.ds(0, 512)],
                    sems_f[buf],
                ).start()
                pltpu.make_async_copy(
                    tab_hbm.at[:, pl.ds(pl.multiple_of(c_lo + 512, 128), 128)],
                    chunks.at[buf, :, pl.ds(512, 128)],
                    sems_f[buf],
                ).start()

        def drain_chunk(c, buf):
            c_lo = jnp.where(c < _BASE_CH, lo + c * _CL, elo)

            @pl.when(c_lo != _PARTIAL)
            def _():
                pltpu.make_async_copy(
                    tab_hbm.at[:, pl.ds(0, _CL)], chunks.at[buf], sems_f[buf]
                ).wait()

            @pl.when(c_lo == _PARTIAL)
            def _():
                pltpu.make_async_copy(
                    tab_hbm.at[:, pl.ds(0, 512)],
                    chunks.at[buf, :, pl.ds(0, 512)],
                    sems_f[buf],
                ).wait()
                pltpu.make_async_copy(
                    tab_hbm.at[:, pl.ds(0, 128)],
                    chunks.at[buf, :, pl.ds(512, 128)],
                    sems_f[buf],
                ).wait()

        for h in (0, 1):
            pltpu.make_async_copy(
                ring.at[h], out_hbm.at[pl.ds(_BATCH, 16)], osems[h]
            ).start()

        def extract_sublist(c, buf, h):
            start = meta[2 * c]

            def do(t):
                av = sa[pl.ds(start + t * 16, 16)]
                pv = sp[pl.ds(start + t * 16, 16)]
                c_lo = jnp.where(c < _BASE_CH, lo + c * _CL, elo)
                cols = jnp.clip(av - c_lo, 0, _CL - 1)
                pltpu.make_async_copy(
                    ring.at[h], out_hbm.at[pl.ds(_BATCH, 16)], osems[h]
                ).wait()
                for k in range(16):
                    col = jnp.full((16,), cols[k], jnp.int32)
                    vals = plsc.load_gather(chunks.at[buf], [lane_iota, col])
                    ring[h, k, :] = vals
                for k in range(16):
                    pltpu.make_async_copy(
                        ring.at[h, k], out_hbm.at[pv[k]], osems[h]
                    ).start()

            return do

        def extract_chunk(c, buf):
            num = meta[2 * c + 1]
            n2 = (num + 31) >> 5

            def eb(tp, _):
                extract_sublist(c, buf, 0)(2 * tp)

                @pl.when(2 * tp + 1 < ((num + 15) >> 4))
                def _():
                    extract_sublist(c, buf, 1)(2 * tp + 1)

                return 0

            lax.fori_loop(0, n2, eb, 0)

        fetch_chunk(jnp.int32(0), 0)

        @pl.loop(0, 16, step=2)
        def _(c):
            @pl.when(c < n_chunks)
            def _():
                @pl.when(c + 1 < n_chunks)
                def _():
                    fetch_chunk(c + 1, 1)

                drain_chunk(c, 0)
                extract_chunk(c, 0)

            @pl.when(c + 1 < n_chunks)
            def _():
                @pl.when(c + 2 < n_chunks)
                def _():
                    fetch_chunk(c + 2, 0)

                drain_chunk(c + 1, 1)
                extract_chunk(c + 1, 1)

        for h in (0, 1):
            pltpu.make_async_copy(
                ring.at[h], out_hbm.at[pl.ds(_BATCH, 16)], osems[h]
            ).wait()

    return run(table_t, idx)


def _sc_gather_t(table_t, idx):
    mesh = plsc.VectorSubcoreMesh(core_axis_name="c", subcore_axis_name="s")

    @functools.partial(
        pl.kernel,
        out_type=jax.ShapeDtypeStruct((_EMBED, _BATCH), table_t.dtype),
        mesh=mesh,
        compiler_params=pltpu.CompilerParams(
            use_tc_tiling_on_sc=True, needs_layout_passes=False
        ),
        scratch_types=[
            pltpu.VMEM((_BPW,), jnp.int32),
            pltpu.VMEM((_NBUF, _EMBED, _G * 128), jnp.float32),
            pltpu.VMEM((_EMBED, _BPW), jnp.float32),
            pltpu.SemaphoreType.DMA,
            pltpu.SemaphoreType.DMA,
            pltpu.SemaphoreType.DMA,
            pltpu.SemaphoreType.DMA,
        ],
    )
    def run(tab_hbm, idx_hbm, out_hbm, idx_v, slabs, acc,
            sem0, sem1, sem2, osem):
        wid = lax.axis_index("s") * 2 + lax.axis_index("c")
        base = wid * _BPW
        pltpu.async_copy(idx_hbm.at[pl.ds(base, _BPW)], idx_v, sem0).wait()

        sems = (sem0, sem1, sem2)
        lane_iota = lax.iota(jnp.int32, 16)

        def fetch_group(g, buf):
            v = idx_v[pl.ds(g * _G, _G)]
            for k in range(_G):
                lane0 = pl.multiple_of((v[k] >> 7) << 7, 128)
                pltpu.make_async_copy(
                    tab_hbm.at[:, pl.ds(lane0, 128)],
                    slabs.at[buf, :, pl.ds(k * 128, 128)],
                    sems[buf],
                ).start()

        def drain_group(buf):
            pltpu.make_async_copy(
                tab_hbm.at[:, pl.ds(0, _G * 128)],
                slabs.at[buf],
                sems[buf],
            ).wait()

        def extract_group(g, buf):
            v = idx_v[pl.ds(g * _G, _G)]
            src_cols = lane_iota * 128 + (v & 127)
            dst_cols = lane_iota + g * _G
            for e in range(_EMBED):
                row = jnp.full((16,), e, jnp.int32)
                vals = plsc.load_gather(slabs.at[buf], [row, src_cols])
                plsc.store_scatter(acc, [row, dst_cols], vals)

        for b in range(_NBUF - 1):
            fetch_group(b, b)

        @pl.loop(0, _NG - 2, step=_NBUF)
        def _(g):
            for b in range(_NBUF):
                fetch_group(g + b + _NBUF - 1, (b + _NBUF - 1) % _NBUF)
                drain_group(b)
                extract_group(g + b, b)

        for b in range(2):
            drain_group(b)
            extract_group(_NG - 2 + b, b)

        pltpu.async_copy(acc, out_hbm.at[:, pl.ds(base, _BPW)], osem).wait()

    return run(table_t, idx)


def _tc_matmul_rnn(state, W_enc, rnn_hxs):
    def body(s_ref, w_ref, r_ref, o_ref, r_out_ref):
        s = s_ref[...].astype(jnp.bfloat16)
        w = w_ref[...].astype(jnp.bfloat16)
        o_ref[...] = lax.dot_general(
            w, s, (((0,), (1,)), ((), ())),
            preferred_element_type=jnp.float32,
        )
        r_out_ref[...] = r_ref[...]

    return pl.pallas_call(
        body,
        grid=(_BATCH // _TB,),
        in_specs=[
            pl.BlockSpec((_TB, _D_STATE), lambda i: (i, 0)),
            pl.BlockSpec((_D_STATE, _D_OUT), lambda i: (0, 0)),
            pl.BlockSpec((_TB, _D_OUT), lambda i: (i, 0)),
        ],
        out_specs=[
            pl.BlockSpec((_D_OUT, _TB), lambda i: (0, i)),
            pl.BlockSpec((_TB, _D_OUT), lambda i: (i, 0)),
        ],
        out_shape=[
            jax.ShapeDtypeStruct((_D_OUT + _EMBED, _BATCH), jnp.float32),
            jax.ShapeDtypeStruct((_BATCH, _D_OUT), jnp.float32),
        ],
        compiler_params=pltpu.CompilerParams(
            dimension_semantics=("parallel",)
        ),
    )(state, W_enc, rnn_hxs)


def _tc_concat_act(out_partial, act_t):
    def body(_, a_ref, o_ref):
        o_ref[...] = a_ref[...].T

    return pl.pallas_call(
        body,
        grid=(_BATCH // _CB,),
        in_specs=[
            pl.BlockSpec(memory_space=pl.ANY),
            pl.BlockSpec((_CB, _EMBED), lambda i: (i, 0)),
        ],
        out_specs=pl.BlockSpec(
            (_EMBED, _CB), lambda i: (_D_OUT // _EMBED, i)
        ),
        out_shape=jax.ShapeDtypeStruct((_D_OUT + _EMBED, _BATCH), jnp.float32),
        input_output_aliases={0: 0},
    )(out_partial, act_t)


def kernel(state, last_action, rnn_hxs, W_enc, table):
    idx = last_action.astype(jnp.int32)
    act = _sc_gather_rows(table.T, idx)[:_BATCH]
    out_partial, rnn_out = _tc_matmul_rnn(state, W_enc, rnn_hxs)
    out_t = _tc_concat_act(out_partial, act)
    return out_t.T, rnn_out


# speedup vs baseline: 2.4785x; 2.4785x over previous
import functools

import jax
import jax.numpy as jnp
from jax import lax
from jax.experimental import pallas as pl
from jax.experimental.pallas import tpu as pltpu
from jax.experimental.pallas import tpu_sc as plsc

_BATCH = 16384
_D_STATE = 512
_D_OUT = 512
_EMBED = 16

_NW = 32
_BPW = _BATCH // _NW
_G = 16
_NG = _BPW // _G
_NBUF = 3

_TB = 1024
_CB = 4096


def _sc_gather_t(table_t, idx):
    mesh = plsc.VectorSubcoreMesh(core_axis_name="c", subcore_axis_name="s")

    @functools.partial(
        pl.kernel,
        out_type=jax.ShapeDtypeStruct((_EMBED, _BATCH), table_t.dtype),
        mesh=mesh,
        compiler_params=pltpu.CompilerParams(
            use_tc_tiling_on_sc=True, needs_layout_passes=False
        ),
        scratch_types=[
            pltpu.VMEM((_BPW,), jnp.int32),
            pltpu.VMEM((_NBUF, _EMBED, _G * 128), jnp.float32),
            pltpu.VMEM((_EMBED, _BPW), jnp.float32),
            pltpu.SemaphoreType.DMA,
            pltpu.SemaphoreType.DMA,
            pltpu.SemaphoreType.DMA,
            pltpu.SemaphoreType.DMA,
        ],
    )
    def run(tab_hbm, idx_hbm, out_hbm, idx_v, slabs, acc,
            sem0, sem1, sem2, osem):
        wid = lax.axis_index("s") * 2 + lax.axis_index("c")
        base = wid * _BPW
        pltpu.async_copy(idx_hbm.at[pl.ds(base, _BPW)], idx_v, sem0).wait()

        sems = (sem0, sem1, sem2)
        lane_iota = lax.iota(jnp.int32, 16)

        def fetch_group(g, buf):
            v = idx_v[pl.ds(g * _G, _G)]
            for k in range(_G):
                lane0 = pl.multiple_of((v[k] >> 7) << 7, 128)
                pltpu.make_async_copy(
                    tab_hbm.at[:, pl.ds(lane0, 128)],
                    slabs.at[buf, :, pl.ds(k * 128, 128)],
                    sems[buf],
                ).start()

        def drain_group(buf):
            pltpu.make_async_copy(
                tab_hbm.at[:, pl.ds(0, _G * 128)],
                slabs.at[buf],
                sems[buf],
            ).wait()

        def extract_group(g, buf):
            v = idx_v[pl.ds(g * _G, _G)]
            src_cols = lane_iota * 128 + (v & 127)
            dst_cols = lane_iota + g * _G
            for e in range(_EMBED):
                row = jnp.full((16,), e, jnp.int32)
                vals = plsc.load_gather(slabs.at[buf], [row, src_cols])
                plsc.store_scatter(acc, [row, dst_cols], vals)

        for b in range(_NBUF - 1):
            fetch_group(b, b)

        @pl.loop(0, _NG - 2, step=_NBUF)
        def _(g):
            for b in range(_NBUF):
                fetch_group(g + b + _NBUF - 1, (b + _NBUF - 1) % _NBUF)
                drain_group(b)
                extract_group(g + b, b)

        for b in range(2):
            drain_group(b)
            extract_group(_NG - 2 + b, b)

        pltpu.async_copy(acc, out_hbm.at[:, pl.ds(base, _BPW)], osem).wait()

    return run(table_t, idx)


def _tc_matmul_rnn(state, W_enc, rnn_hxs):
    def body(s_ref, w_ref, r_ref, o_ref, r_out_ref):
        s = s_ref[...].astype(jnp.bfloat16)
        w = w_ref[...].astype(jnp.bfloat16)
        o_ref[...] = lax.dot_general(
            w, s, (((0,), (1,)), ((), ())),
            preferred_element_type=jnp.float32,
        )
        r_out_ref[...] = r_ref[...]

    return pl.pallas_call(
        body,
        grid=(_BATCH // _TB,),
        in_specs=[
            pl.BlockSpec((_TB, _D_STATE), lambda i: (i, 0)),
            pl.BlockSpec((_D_STATE, _D_OUT), lambda i: (0, 0)),
            pl.BlockSpec((_TB, _D_OUT), lambda i: (i, 0)),
        ],
        out_specs=[
            pl.BlockSpec((_D_OUT, _TB), lambda i: (0, i)),
            pl.BlockSpec((_TB, _D_OUT), lambda i: (i, 0)),
        ],
        out_shape=[
            jax.ShapeDtypeStruct((_D_OUT + _EMBED, _BATCH), jnp.float32),
            jax.ShapeDtypeStruct((_BATCH, _D_OUT), jnp.float32),
        ],
        compiler_params=pltpu.CompilerParams(
            dimension_semantics=("parallel",)
        ),
    )(state, W_enc, rnn_hxs)


def _tc_concat_act(out_partial, act_t):
    def body(_, a_ref, o_ref):
        o_ref[...] = a_ref[...]

    return pl.pallas_call(
        body,
        grid=(_BATCH // _CB,),
        in_specs=[
            pl.BlockSpec(memory_space=pl.ANY),
            pl.BlockSpec((_EMBED, _CB), lambda i: (0, i)),
        ],
        out_specs=pl.BlockSpec(
            (_EMBED, _CB), lambda i: (_D_OUT // _EMBED, i)
        ),
        out_shape=jax.ShapeDtypeStruct((_D_OUT + _EMBED, _BATCH), jnp.float32),
        input_output_aliases={0: 0},
    )(out_partial, act_t)


def kernel(state, last_action, rnn_hxs, W_enc, table):
    idx = last_action.astype(jnp.int32)
    act_t = _sc_gather_t(table.T, idx)
    out_partial, rnn_out = _tc_matmul_rnn(state, W_enc, rnn_hxs)
    out_t = _tc_concat_act(out_partial, act_t)
    return out_t.T, rnn_out
